# Initial kernel scaffold; baseline (speedup 1.0000x reference)
import jax
import jax.numpy as jnp
from jax.experimental import pallas as pl


def kernel(dfn, dfn_fa, tokenized_parts_latent, encoding, W, b):
    raise NotImplementedError("write your pallas kernel here")


# speedup vs baseline: 4.3456x; 4.3456x over previous
import functools

import jax
import jax.numpy as jnp
from jax import lax
from jax.experimental import pallas as pl
from jax.experimental.pallas import tpu as pltpu
from jax.experimental.pallas import tpu_sc as plsc

D = 64
CHUNK = 128
NWORKERS = 32
LANES = 16


def _sc_gather_add(encoding, idx1, idx2):
    rows = idx1.shape[0]
    rows_per_w = rows // NWORKERS
    nch = rows_per_w // CHUNK
    mesh = plsc.VectorSubcoreMesh(core_axis_name="c", subcore_axis_name="s")

    @functools.partial(
        pl.kernel,
        mesh=mesh,
        out_type=jax.ShapeDtypeStruct((rows, D), jnp.float32),
        scratch_types=[
            pltpu.VMEM((rows_per_w,), jnp.int32),
            pltpu.VMEM((rows_per_w,), jnp.int32),
            pltpu.VMEM((CHUNK, D), jnp.float32),
            pltpu.VMEM((CHUNK, D), jnp.float32),
            pltpu.SemaphoreType.DMA,
            pltpu.SemaphoreType.DMA,
        ],
        compiler_params=pltpu.CompilerParams(use_tc_tiling_on_sc=False),
    )
    def k(enc_hbm, i1_hbm, i2_hbm, out_hbm, i1_v, i2_v, r1_v, r2_v, sem1, sem2):
        wid = lax.axis_index("s") * 2 + lax.axis_index("c")
        rbase = wid * rows_per_w
        pltpu.sync_copy(i1_hbm.at[pl.ds(rbase, rows_per_w)], i1_v)
        pltpu.sync_copy(i2_hbm.at[pl.ds(rbase, rows_per_w)], i2_v)

        def chunk_body(j, _):
            isl = pl.ds(j * CHUNK, CHUNK)
            cp1 = pltpu.async_copy(enc_hbm.at[i1_v.at[isl]], r1_v, sem1)
            cp2 = pltpu.async_copy(enc_hbm.at[i2_v.at[isl]], r2_v, sem2)
            cp1.wait()
            cp2.wait()

            def add_body(i, _):
                for c in range(D // LANES):
                    sl = pl.ds(c * LANES, LANES)
                    r1_v[i, sl] = r1_v[i, sl] + r2_v[i, sl]
                return 0

            lax.fori_loop(0, CHUNK, add_body, 0)
            row_start = rbase + j * CHUNK
            pltpu.sync_copy(r1_v, out_hbm.at[pl.ds(row_start, CHUNK)])
            return 0

        lax.fori_loop(0, nch, chunk_body, 0)

    return k(encoding, idx1, idx2)


def _tc_body(pe_ref, lat_ref, w1_ref, w2_ref, b_ref, out_ref):
    acc = jnp.dot(pe_ref[...], w1_ref[...], preferred_element_type=jnp.float32)
    acc += jnp.dot(lat_ref[...], w2_ref[...], preferred_element_type=jnp.float32)
    out_ref[...] = acc + b_ref[...]


def _tc_combine(pe2, lat2, w1d, w2d, b2):
    rows2 = pe2.shape[0]
    blk = 2048
    grid = (rows2 // blk,)
    return pl.pallas_call(
        _tc_body,
        grid=grid,
        in_specs=[
            pl.BlockSpec((blk, 2 * D), lambda i: (i, 0)),
            pl.BlockSpec((blk, 2 * D), lambda i: (i, 0)),
            pl.BlockSpec((2 * D, 2 * D), lambda i: (0, 0)),
            pl.BlockSpec((2 * D, 2 * D), lambda i: (0, 0)),
            pl.BlockSpec((1, 2 * D), lambda i: (0, 0)),
        ],
        out_specs=pl.BlockSpec((blk, 2 * D), lambda i: (i, 0)),
        out_shape=jax.ShapeDtypeStruct((rows2, 2 * D), jnp.float32),
        compiler_params=pltpu.CompilerParams(
            dimension_semantics=("arbitrary",),
        ),
    )(pe2, lat2, w1d, w2d, b2)


def kernel(dfn, dfn_fa, tokenized_parts_latent, encoding, W, b):
    nb, npart, dm = tokenized_parts_latent.shape
    rows = nb * npart
    idx1 = dfn.reshape(-1).astype(jnp.int32)
    idx2 = dfn_fa.reshape(-1).astype(jnp.int32)
    pe = _sc_gather_add(encoding, idx1, idx2)

    pe2 = pe.reshape(rows // 2, 2 * dm)
    lat2 = tokenized_parts_latent.reshape(rows // 2, 2 * dm)
    w1t = W[:, :dm].T
    w2t = W[:, dm:].T
    zero = jnp.zeros((dm, dm), jnp.float32)
    w1d = jnp.block([[w1t, zero], [zero, w1t]])
    w2d = jnp.block([[w2t, zero], [zero, w2t]])
    b2 = jnp.tile(b.reshape(1, dm), (1, 2))
    out2 = _tc_combine(pe2, lat2, w1d, w2d, b2)
    return out2.reshape(nb, npart, dm)
